# baseline (speedup 1.0000x reference)
import functools
import math

import jax
import jax.numpy as jnp
from jax import lax
from jax.experimental import pallas as pl
from jax.experimental.pallas import tpu as pltpu
from jax.experimental.pallas import tpu_sc as plsc

N = 50000
E = 800000
HID = 64
DH = 32
NSUB = 16
CH = 80
EPS = E // NSUB
NCH = EPS // CH
RPT = 3200
RPT_LAST = N - 15 * RPT
ZR = 200
CW = 16
ECS = E // 2 // NSUB
NCC = 312
CTL = 40
ROW_BLK = 5000
DH2 = DH // 2
_BN = 1.0 / math.sqrt(1.0 + 1e-5)


def _elu(v):
    return jnp.where(v > 0.0, v, jnp.exp(v) - 1.0)


def _split_tab(t, ref):
    u = lax.bitcast_convert_type(t, jnp.int32)
    bf = ((u + 0x7FFF + ((u >> 16) & 1)) >> 16) & 0xFFFF
    for half in (0, 1):
        lo = bf[:, half * DH:half * DH + DH2]
        hi = bf[:, half * DH + DH2:(half + 1) * DH]
        ref[half, :, :] = lo | (hi << 16)


def _enc_body(x_ref, w1, b1, w2, b2, wa, ba, wb, h_ref, a_ref, bt_ref):
    h1 = _elu(jnp.dot(x_ref[...], w1[...],
                      preferred_element_type=jnp.float32) + b1[...])
    h = _elu(jnp.dot(h1, w2[...],
                     preferred_element_type=jnp.float32) + b2[...])
    h_ref[...] = h
    _split_tab(jnp.dot(h, wa[...], preferred_element_type=jnp.float32) + ba[...],
               a_ref)
    _split_tab(jnp.dot(h, wb[...], preferred_element_type=jnp.float32), bt_ref)


def _tc_encode(x, w1, b1, w2, b2, wa, ba, wb):
    grid = (N // ROW_BLK,)
    full = lambda shape: pl.BlockSpec(shape, lambda i: (0, 0))
    return pl.pallas_call(
        _enc_body,
        grid=grid,
        in_specs=[
            pl.BlockSpec((ROW_BLK, 16), lambda i: (i, 0)),
            full((16, HID)), full((1, HID)), full((HID, HID)), full((1, HID)),
            full((HID, HID)), full((1, HID)), full((HID, HID)),
        ],
        out_specs=[
            pl.BlockSpec((ROW_BLK, HID), lambda i: (i, 0)),
            pl.BlockSpec((2, ROW_BLK, DH2), lambda i: (0, i, 0)),
            pl.BlockSpec((2, ROW_BLK, DH2), lambda i: (0, i, 0)),
        ],
        out_shape=[
            jax.ShapeDtypeStruct((N, HID), jnp.float32),
            jax.ShapeDtypeStruct((2, N, DH2), jnp.int32),
            jax.ShapeDtypeStruct((2, N, DH2), jnp.int32),
        ],
    )(x, w1, b1, w2, b2, wa, ba, wb)


def _combine(agg_ref, cnt_ref, h, sg, be):
    agg = jnp.concatenate([agg_ref[0, :, :], agg_ref[1, :, :]], axis=1)
    c0 = cnt_ref[0, :, 0:1] + cnt_ref[1, :, 0:1]
    inv = 1.0 / jnp.maximum(c0, 1.0)
    ind = (c0 > 0.0).astype(jnp.float32)
    return agg * inv * sg + be * ind + h


def _comb_body(agg_ref, cnt_ref, h_ref, sg, be, wa, ba, wb,
               hn_ref, a_ref, bt_ref):
    hn = _combine(agg_ref, cnt_ref, h_ref[...], sg[...], be[...])
    hn_ref[...] = hn
    _split_tab(jnp.dot(hn, wa[...], preferred_element_type=jnp.float32) + ba[...],
               a_ref)
    _split_tab(jnp.dot(hn, wb[...], preferred_element_type=jnp.float32), bt_ref)


def _tc_combine(agg, cnt, h, sg, be, wa, ba, wb):
    grid = (N // ROW_BLK,)
    full = lambda shape: pl.BlockSpec(shape, lambda i: (0, 0))
    row = lambda w: pl.BlockSpec((ROW_BLK, w), lambda i: (i, 0))
    return pl.pallas_call(
        _comb_body,
        grid=grid,
        in_specs=[pl.BlockSpec((2, ROW_BLK, DH), lambda i: (0, i, 0)),
                  pl.BlockSpec((2, ROW_BLK, CW), lambda i: (0, i, 0)),
                  row(HID),
                  full((1, HID)), full((1, HID)),
                  full((HID, HID)), full((1, HID)), full((HID, HID))],
        out_specs=[row(HID),
                   pl.BlockSpec((2, ROW_BLK, DH2), lambda i: (0, i, 0)),
                   pl.BlockSpec((2, ROW_BLK, DH2), lambda i: (0, i, 0))],
        out_shape=[jax.ShapeDtypeStruct((N, HID), jnp.float32),
                   jax.ShapeDtypeStruct((2, N, DH2), jnp.int32),
                   jax.ShapeDtypeStruct((2, N, DH2), jnp.int32)],
    )(agg, cnt, h, sg, be, wa, ba, wb)


def _final_body(agg_ref, cnt_ref, h_ref, sg, be,
                w1, b1, w2, b2, w3, b3, o_ref):
    hn = _combine(agg_ref, cnt_ref, h_ref[...], sg[...], be[...])
    o = _elu(jnp.dot(hn, w1[...], preferred_element_type=jnp.float32) + b1[...])
    o = _elu(jnp.dot(o, w2[...], preferred_element_type=jnp.float32) + b2[...])
    o_ref[...] = jnp.dot(o, w3[...], preferred_element_type=jnp.float32) + b3[...]


def _tc_final(agg, cnt, h, sg, be, w1, b1, w2, b2, w3, b3):
    grid = (N // ROW_BLK,)
    full = lambda shape: pl.BlockSpec(shape, lambda i: (0, 0))
    row = lambda w: pl.BlockSpec((ROW_BLK, w), lambda i: (i, 0))
    return pl.pallas_call(
        _final_body,
        grid=grid,
        in_specs=[pl.BlockSpec((2, ROW_BLK, DH), lambda i: (0, i, 0)),
                  pl.BlockSpec((2, ROW_BLK, CW), lambda i: (0, i, 0)),
                  row(HID),
                  full((1, HID)), full((1, HID)),
                  full((HID, 64)), full((1, 64)),
                  full((64, 32)), full((1, 32)),
                  full((32, 8)), full((1, 8))],
        out_specs=[row(8)],
        out_shape=[jax.ShapeDtypeStruct((N, 8), jnp.float32)],
    )(agg, cnt, h, sg, be, w1, b1, w2, b2, w3, b3)[0]


_SC_MESH = plsc.VectorSubcoreMesh(core_axis_name="c", subcore_axis_name="s")
_SC_PARAMS = pltpu.CompilerParams(use_tc_tiling_on_sc=False)


def _sc_readback(sh, out, s):
    @pl.when(s < 15)
    def _():
        pltpu.sync_copy(sh.at[pl.ds(s * RPT, RPT)], out.at[pl.ds(s * RPT, RPT)])

    @pl.when(s == 15)
    def _():
        pltpu.sync_copy(sh.at[pl.ds(15 * RPT, RPT_LAST)],
                        out.at[pl.ds(15 * RPT, RPT_LAST)])


SB = 25
NSB = NCH // SB
SLOTS = 5
LOOKAHEAD = 3


@functools.partial(
    pl.kernel, mesh=_SC_MESH, compiler_params=_SC_PARAMS,
    out_type=[jax.ShapeDtypeStruct((2, N, DH), jnp.float32)],
    scratch_types=[
        pltpu.VMEM_SHARED((N, DH), jnp.float32),
        pltpu.VMEM((SB * CH,), jnp.int32),
        pltpu.VMEM((SB * CH,), jnp.int32),
        pltpu.VMEM((SLOTS, CH), jnp.int32),
        pltpu.VMEM((SLOTS, CH, DH2), jnp.int32),
        pltpu.VMEM((SLOTS, CH, DH2), jnp.int32),
        pltpu.VMEM((SLOTS, CH, DH), jnp.float32),
        pltpu.SemaphoreType.DMA((SLOTS,)),
        pltpu.SemaphoreType.DMA((SLOTS,)),
    ])
def _sc_edge(a3, b3, row_h, col_h, zc32, agg_out,
             agg_sh, idxr, idxc, sidx, a_buf, b_buf, val, sem_g, sem_s):
    c = lax.axis_index("c")
    s = lax.axis_index("s")
    a2 = a3.at[c]
    b2 = b3.at[c]

    @pl.when(s < 15)
    def _():
        pltpu.sync_copy(zc32.at[pl.ds(s * RPT, RPT)],
                        agg_sh.at[pl.ds(s * RPT, RPT)])

    @pl.when(s == 15)
    def _():
        pltpu.sync_copy(zc32.at[pl.ds(15 * RPT, RPT_LAST)],
                        agg_sh.at[pl.ds(15 * RPT, RPT_LAST)])

    plsc.subcore_barrier()

    def wait_scatter(b):
        pltpu.make_async_copy(val.at[b], agg_sh.at[sidx.at[b]],
                              sem_s.at[b]).wait()

    def issue_gather(k, sb):
        b = k % SLOTS
        if k < SLOTS:
            @pl.when(sb > 0)
            def _():
                wait_scatter(b)
        else:
            wait_scatter(b)
        pltpu.async_copy(a2.at[idxr.at[pl.ds(k * CH, CH)]], a_buf.at[b],
                         sem_g.at[b])
        pltpu.async_copy(b2.at[idxc.at[pl.ds(k * CH, CH)]], b_buf.at[b],
                         sem_g.at[b])

    def wait_gather(k, b):
        pltpu.make_async_copy(a2.at[idxr.at[pl.ds(k * CH, CH)]], a_buf.at[b],
                              sem_g.at[b]).wait()
        pltpu.make_async_copy(b2.at[idxc.at[pl.ds(k * CH, CH)]], b_buf.at[b],
                              sem_g.at[b]).wait()

    ebase = s * EPS

    def superblock(sb, carry):
        base = ebase + sb * (SB * CH)
        pltpu.sync_copy(row_h.at[pl.ds(base, SB * CH)], idxr)
        pltpu.sync_copy(col_h.at[pl.ds(base, SB * CH)], idxc)

        for k in range(LOOKAHEAD):
            issue_gather(k, sb)

        for k in range(SB):
            b = k % SLOTS
            wait_gather(k, b)
            for i in range(CH // 16):
                sl = pl.ds(i * 16, 16)
                sidx[b, sl] = idxr[pl.ds(k * CH + i * 16, 16)]

            hi_mask = jnp.full((16,), -65536, jnp.int32)

            def crow(q, cc):
                r0 = q * 8
                for u in range(8):
                    r = r0 + u
                    aw = a_buf[b, r, pl.ds(0, DH2)]
                    bw = b_buf[b, r, pl.ds(0, DH2)]
                    v0 = (lax.bitcast_convert_type(aw << 16, jnp.float32)
                          + lax.bitcast_convert_type(bw << 16, jnp.float32))
                    v1 = (lax.bitcast_convert_type(aw & hi_mask, jnp.float32)
                          + lax.bitcast_convert_type(bw & hi_mask, jnp.float32))
                    val[b, r, pl.ds(0, 16)] = jnp.where(
                        v0 > 0.0, v0, jnp.exp(v0) - 1.0)
                    val[b, r, pl.ds(16, 16)] = jnp.where(
                        v1 > 0.0, v1, jnp.exp(v1) - 1.0)
                return cc
            lax.fori_loop(0, CH // 8, crow, 0)

            pltpu.async_copy(val.at[b], agg_sh.at[sidx.at[b]],
                             sem_s.at[b], add=True)
            if k + LOOKAHEAD < SB:
                issue_gather(k + LOOKAHEAD, sb)
        return carry
    lax.fori_loop(0, NSB, superblock, 0)

    for b in range(SLOTS):
        wait_scatter(b)

    plsc.subcore_barrier()
    _sc_readback(agg_sh, agg_out.at[c], s)


NROWS2D = E // CH
WROWS = NROWS2D // 32
WV = 8
NWV = WROWS // WV
NEXTRA = NROWS2D - 32 * WROWS


@functools.partial(
    pl.kernel, mesh=_SC_MESH, compiler_params=_SC_PARAMS,
    out_type=[jax.ShapeDtypeStruct((2, N, CW), jnp.float32)],
    scratch_types=[
        pltpu.VMEM_SHARED((N, CW), jnp.float32),
        pltpu.VMEM((WV, CH), jnp.int32),
        pltpu.VMEM((1, CH), jnp.int32),
        pltpu.VMEM((CH, CW), jnp.float32),
        pltpu.SemaphoreType.DMA((WV,)),
    ])
def _sc_count(row2d, zc, ones_h, cnt_out,
              cnt_sh, wav, wav1, ones_b, sem_w):
    c = lax.axis_index("c")
    s = lax.axis_index("s")
    w = c * NSUB + s

    @pl.when(s < 15)
    def _():
        pltpu.sync_copy(zc.at[pl.ds(s * RPT, RPT)],
                        cnt_sh.at[pl.ds(s * RPT, RPT)])

    @pl.when(s == 15)
    def _():
        pltpu.sync_copy(zc.at[pl.ds(15 * RPT, RPT_LAST)],
                        cnt_sh.at[pl.ds(15 * RPT, RPT_LAST)])
    pltpu.sync_copy(ones_h, ones_b)

    plsc.subcore_barrier()

    base = w * WROWS

    def wave(v, carry):
        pltpu.sync_copy(row2d.at[pl.ds(base + v * WV, WV)], wav)
        for j in range(WV):
            pltpu.async_copy(ones_b, cnt_sh.at[wav.at[j]],
                             sem_w.at[j], add=True)
        for j in range(WV):
            pltpu.make_async_copy(ones_b, cnt_sh.at[wav.at[j]],
                                  sem_w.at[j]).wait()
        return carry
    lax.fori_loop(0, NWV, wave, 0)

    @pl.when(w < NEXTRA)
    def _():
        pltpu.sync_copy(row2d.at[pl.ds(32 * WROWS + w, 1)], wav1)
        pltpu.sync_copy(ones_b, cnt_sh.at[wav1.at[0]], add=True)

    plsc.subcore_barrier()
    _sc_readback(cnt_sh, cnt_out.at[c], s)


def kernel(x, edge_index, batch, params):
    row = edge_index[0]
    col = edge_index[1]

    wa = [params[f'W_c{l}'][:HID] - params[f'W_c{l}'][HID:] for l in range(4)]
    wb = [params[f'W_c{l}'][HID:] for l in range(4)]
    ba = [params[f'b_c{l}'].reshape(1, HID) for l in range(4)]
    sg = [(params[f'g_c{l}'] * _BN).reshape(1, HID) for l in range(4)]
    be = [params[f'be_c{l}'].reshape(1, HID) for l in range(4)]

    h, a_tab, b_tab = _tc_encode(
        x, params['W_lc1'], params['b_lc1'].reshape(1, HID),
        params['W_lc2'], params['b_lc2'].reshape(1, HID),
        wa[0], ba[0], wb[0])

    zc = jnp.zeros((N, CW), jnp.float32)
    ones = jnp.ones((CH, CW), jnp.float32)
    (cnt2,) = _sc_count(row.reshape(NROWS2D, CH), zc, ones)

    zc32 = jnp.zeros((N, DH), jnp.float32)
    (agg2,) = _sc_edge(a_tab, b_tab, row, col, zc32)

    for l in range(1, 4):
        h, a_tab, b_tab = _tc_combine(agg2, cnt2, h,
                                      sg[l - 1], be[l - 1],
                                      wa[l], ba[l], wb[l])
        (agg2,) = _sc_edge(a_tab, b_tab, row, col, zc32)

    o = _tc_final(agg2, cnt2, h, sg[3], be[3],
                  params['W_o1'], params['b_o1'].reshape(1, 64),
                  params['W_o2'], params['b_o2'].reshape(1, 32),
                  params['W_o3'], params['b_o3'].reshape(1, 8))
    return (o, batch)


# speedup vs baseline: 2.0003x; 2.0003x over previous
import functools
import math

import jax
import jax.numpy as jnp
from jax import lax
from jax.experimental import pallas as pl
from jax.experimental.pallas import tpu as pltpu
from jax.experimental.pallas import tpu_sc as plsc

N = 50000
E = 800000
HID = 64
DH = 32
NSUB = 16
CH = 80
EPS = E // NSUB
NCH = EPS // CH
RPT = 3200
RPT_LAST = N - 15 * RPT
ZR = 200
CW = 16
ECS = E // 2 // NSUB
NCC = 312
CTL = 40
ROW_BLK = 5000
_BN = 1.0 / math.sqrt(1.0 + 1e-5)


def _elu(v):
    return jnp.where(v > 0.0, v, jnp.exp(v) - 1.0)


def _split_tab(t, ref):
    ref[0, :, :] = t[:, :DH]
    ref[1, :, :] = t[:, DH:]


def _enc_body(x_ref, w1, b1, w2, b2, wa, ba, wb, h_ref, a_ref, bt_ref):
    h1 = _elu(jnp.dot(x_ref[...], w1[...],
                      preferred_element_type=jnp.float32) + b1[...])
    h = _elu(jnp.dot(h1, w2[...],
                     preferred_element_type=jnp.float32) + b2[...])
    h_ref[...] = h
    _split_tab(jnp.dot(h, wa[...], preferred_element_type=jnp.float32) + ba[...],
               a_ref)
    _split_tab(jnp.dot(h, wb[...], preferred_element_type=jnp.float32), bt_ref)


def _tc_encode(x, w1, b1, w2, b2, wa, ba, wb):
    grid = (N // ROW_BLK,)
    full = lambda shape: pl.BlockSpec(shape, lambda i: (0, 0))
    return pl.pallas_call(
        _enc_body,
        grid=grid,
        in_specs=[
            pl.BlockSpec((ROW_BLK, 16), lambda i: (i, 0)),
            full((16, HID)), full((1, HID)), full((HID, HID)), full((1, HID)),
            full((HID, HID)), full((1, HID)), full((HID, HID)),
        ],
        out_specs=[
            pl.BlockSpec((ROW_BLK, HID), lambda i: (i, 0)),
            pl.BlockSpec((2, ROW_BLK, DH), lambda i: (0, i, 0)),
            pl.BlockSpec((2, ROW_BLK, DH), lambda i: (0, i, 0)),
        ],
        out_shape=[
            jax.ShapeDtypeStruct((N, HID), jnp.float32),
            jax.ShapeDtypeStruct((2, N, DH), jnp.float32),
            jax.ShapeDtypeStruct((2, N, DH), jnp.float32),
        ],
    )(x, w1, b1, w2, b2, wa, ba, wb)


def _combine(agg_ref, cnt_ref, h, sg, be):
    agg = jnp.concatenate([agg_ref[0, :, :], agg_ref[1, :, :]], axis=1)
    c0 = cnt_ref[0, :, 0:1] + cnt_ref[1, :, 0:1]
    inv = 1.0 / jnp.maximum(c0, 1.0)
    ind = (c0 > 0.0).astype(jnp.float32)
    return agg * inv * sg + be * ind + h


def _comb_body(agg_ref, cnt_ref, h_ref, sg, be, wa, ba, wb,
               hn_ref, a_ref, bt_ref):
    hn = _combine(agg_ref, cnt_ref, h_ref[...], sg[...], be[...])
    hn_ref[...] = hn
    _split_tab(jnp.dot(hn, wa[...], preferred_element_type=jnp.float32) + ba[...],
               a_ref)
    _split_tab(jnp.dot(hn, wb[...], preferred_element_type=jnp.float32), bt_ref)


def _tc_combine(agg, cnt, h, sg, be, wa, ba, wb):
    grid = (N // ROW_BLK,)
    full = lambda shape: pl.BlockSpec(shape, lambda i: (0, 0))
    row = lambda w: pl.BlockSpec((ROW_BLK, w), lambda i: (i, 0))
    return pl.pallas_call(
        _comb_body,
        grid=grid,
        in_specs=[pl.BlockSpec((2, ROW_BLK, DH), lambda i: (0, i, 0)),
                  pl.BlockSpec((2, ROW_BLK, CW), lambda i: (0, i, 0)),
                  row(HID),
                  full((1, HID)), full((1, HID)),
                  full((HID, HID)), full((1, HID)), full((HID, HID))],
        out_specs=[row(HID),
                   pl.BlockSpec((2, ROW_BLK, DH), lambda i: (0, i, 0)),
                   pl.BlockSpec((2, ROW_BLK, DH), lambda i: (0, i, 0))],
        out_shape=[jax.ShapeDtypeStruct((N, HID), jnp.float32),
                   jax.ShapeDtypeStruct((2, N, DH), jnp.float32),
                   jax.ShapeDtypeStruct((2, N, DH), jnp.float32)],
    )(agg, cnt, h, sg, be, wa, ba, wb)


def _final_body(agg_ref, cnt_ref, h_ref, sg, be,
                w1, b1, w2, b2, w3, b3, o_ref):
    hn = _combine(agg_ref, cnt_ref, h_ref[...], sg[...], be[...])
    o = _elu(jnp.dot(hn, w1[...], preferred_element_type=jnp.float32) + b1[...])
    o = _elu(jnp.dot(o, w2[...], preferred_element_type=jnp.float32) + b2[...])
    o_ref[...] = jnp.dot(o, w3[...], preferred_element_type=jnp.float32) + b3[...]


def _tc_final(agg, cnt, h, sg, be, w1, b1, w2, b2, w3, b3):
    grid = (N // ROW_BLK,)
    full = lambda shape: pl.BlockSpec(shape, lambda i: (0, 0))
    row = lambda w: pl.BlockSpec((ROW_BLK, w), lambda i: (i, 0))
    return pl.pallas_call(
        _final_body,
        grid=grid,
        in_specs=[pl.BlockSpec((2, ROW_BLK, DH), lambda i: (0, i, 0)),
                  pl.BlockSpec((2, ROW_BLK, CW), lambda i: (0, i, 0)),
                  row(HID),
                  full((1, HID)), full((1, HID)),
                  full((HID, 64)), full((1, 64)),
                  full((64, 32)), full((1, 32)),
                  full((32, 8)), full((1, 8))],
        out_specs=[row(8)],
        out_shape=[jax.ShapeDtypeStruct((N, 8), jnp.float32)],
    )(agg, cnt, h, sg, be, w1, b1, w2, b2, w3, b3)[0]


_SC_MESH = plsc.VectorSubcoreMesh(core_axis_name="c", subcore_axis_name="s")
_SC_PARAMS = pltpu.CompilerParams(use_tc_tiling_on_sc=False)


def _sc_readback(sh, out, s):
    @pl.when(s < 15)
    def _():
        pltpu.sync_copy(sh.at[pl.ds(s * RPT, RPT)], out.at[pl.ds(s * RPT, RPT)])

    @pl.when(s == 15)
    def _():
        pltpu.sync_copy(sh.at[pl.ds(15 * RPT, RPT_LAST)],
                        out.at[pl.ds(15 * RPT, RPT_LAST)])


SB = 25
NSB = NCH // SB
SLOTS = 5
LOOKAHEAD = 3


@functools.partial(
    pl.kernel, mesh=_SC_MESH, compiler_params=_SC_PARAMS,
    out_type=[jax.ShapeDtypeStruct((2, N, DH), jnp.float32)],
    scratch_types=[
        pltpu.VMEM_SHARED((N, DH), jnp.float32),
        pltpu.VMEM((SB * CH,), jnp.int32),
        pltpu.VMEM((SB * CH,), jnp.int32),
        pltpu.VMEM((SLOTS, CH), jnp.int32),
        pltpu.VMEM((SLOTS, CH, DH), jnp.float32),
        pltpu.VMEM((SLOTS, CH, DH), jnp.float32),
        pltpu.SemaphoreType.DMA((SLOTS,)),
        pltpu.SemaphoreType.DMA((SLOTS,)),
    ])
def _sc_edge(a3, b3, row_h, col_h, zc32, agg_out,
             agg_sh, idxr, idxc, sidx, a_buf, b_buf, sem_g, sem_s):
    c = lax.axis_index("c")
    s = lax.axis_index("s")
    a2 = a3.at[c]
    b2 = b3.at[c]

    @pl.when(s < 15)
    def _():
        pltpu.sync_copy(zc32.at[pl.ds(s * RPT, RPT)],
                        agg_sh.at[pl.ds(s * RPT, RPT)])

    @pl.when(s == 15)
    def _():
        pltpu.sync_copy(zc32.at[pl.ds(15 * RPT, RPT_LAST)],
                        agg_sh.at[pl.ds(15 * RPT, RPT_LAST)])

    plsc.subcore_barrier()

    def wait_scatter(b):
        pltpu.make_async_copy(a_buf.at[b], agg_sh.at[sidx.at[b]],
                              sem_s.at[b]).wait()

    def issue_gather(k, sb):
        b = k % SLOTS
        if k < SLOTS:
            @pl.when(sb > 0)
            def _():
                wait_scatter(b)
        else:
            wait_scatter(b)
        pltpu.async_copy(a2.at[idxr.at[pl.ds(k * CH, CH)]], a_buf.at[b],
                         sem_g.at[b])
        pltpu.async_copy(b2.at[idxc.at[pl.ds(k * CH, CH)]], b_buf.at[b],
                         sem_g.at[b])

    def wait_gather(k, b):
        pltpu.make_async_copy(a2.at[idxr.at[pl.ds(k * CH, CH)]], a_buf.at[b],
                              sem_g.at[b]).wait()
        pltpu.make_async_copy(b2.at[idxc.at[pl.ds(k * CH, CH)]], b_buf.at[b],
                              sem_g.at[b]).wait()

    ebase = s * EPS

    def superblock(sb, carry):
        base = ebase + sb * (SB * CH)
        pltpu.sync_copy(row_h.at[pl.ds(base, SB * CH)], idxr)
        pltpu.sync_copy(col_h.at[pl.ds(base, SB * CH)], idxc)

        for k in range(LOOKAHEAD):
            issue_gather(k, sb)

        for k in range(SB):
            b = k % SLOTS
            wait_gather(k, b)
            for i in range(CH // 16):
                sl = pl.ds(i * 16, 16)
                sidx[b, sl] = idxr[pl.ds(k * CH + i * 16, 16)]

            def crow(q, cc):
                r0 = q * 8
                for u in range(8):
                    for j in range(2):
                        sl = pl.ds(j * 16, 16)
                        v = a_buf[b, r0 + u, sl] + b_buf[b, r0 + u, sl]
                        a_buf[b, r0 + u, sl] = jnp.where(
                            v > 0.0, v, jnp.exp(v) - 1.0)
                return cc
            lax.fori_loop(0, CH // 8, crow, 0)

            pltpu.async_copy(a_buf.at[b], agg_sh.at[sidx.at[b]],
                             sem_s.at[b], add=True)
            if k + LOOKAHEAD < SB:
                issue_gather(k + LOOKAHEAD, sb)
        return carry
    lax.fori_loop(0, NSB, superblock, 0)

    for b in range(SLOTS):
        wait_scatter(b)

    plsc.subcore_barrier()
    _sc_readback(agg_sh, agg_out.at[c], s)


NROWS2D = E // CH
WROWS = NROWS2D // 32
WV = 8
NWV = WROWS // WV
NEXTRA = NROWS2D - 32 * WROWS


@functools.partial(
    pl.kernel, mesh=_SC_MESH, compiler_params=_SC_PARAMS,
    out_type=[jax.ShapeDtypeStruct((2, N, CW), jnp.float32)],
    scratch_types=[
        pltpu.VMEM_SHARED((N, CW), jnp.float32),
        pltpu.VMEM((WV, CH), jnp.int32),
        pltpu.VMEM((1, CH), jnp.int32),
        pltpu.VMEM((CH, CW), jnp.float32),
        pltpu.SemaphoreType.DMA((WV,)),
    ])
def _sc_count(row2d, zc, ones_h, cnt_out,
              cnt_sh, wav, wav1, ones_b, sem_w):
    c = lax.axis_index("c")
    s = lax.axis_index("s")
    w = c * NSUB + s

    @pl.when(s < 15)
    def _():
        pltpu.sync_copy(zc.at[pl.ds(s * RPT, RPT)],
                        cnt_sh.at[pl.ds(s * RPT, RPT)])

    @pl.when(s == 15)
    def _():
        pltpu.sync_copy(zc.at[pl.ds(15 * RPT, RPT_LAST)],
                        cnt_sh.at[pl.ds(15 * RPT, RPT_LAST)])
    pltpu.sync_copy(ones_h, ones_b)

    plsc.subcore_barrier()

    base = w * WROWS

    def wave(v, carry):
        pltpu.sync_copy(row2d.at[pl.ds(base + v * WV, WV)], wav)
        for j in range(WV):
            pltpu.async_copy(ones_b, cnt_sh.at[wav.at[j]],
                             sem_w.at[j], add=True)
        for j in range(WV):
            pltpu.make_async_copy(ones_b, cnt_sh.at[wav.at[j]],
                                  sem_w.at[j]).wait()
        return carry
    lax.fori_loop(0, NWV, wave, 0)

    @pl.when(w < NEXTRA)
    def _():
        pltpu.sync_copy(row2d.at[pl.ds(32 * WROWS + w, 1)], wav1)
        pltpu.sync_copy(ones_b, cnt_sh.at[wav1.at[0]], add=True)

    plsc.subcore_barrier()
    _sc_readback(cnt_sh, cnt_out.at[c], s)


def kernel(x, edge_index, batch, params):
    row = edge_index[0]
    col = edge_index[1]

    wa = [params[f'W_c{l}'][:HID] - params[f'W_c{l}'][HID:] for l in range(4)]
    wb = [params[f'W_c{l}'][HID:] for l in range(4)]
    ba = [params[f'b_c{l}'].reshape(1, HID) for l in range(4)]
    sg = [(params[f'g_c{l}'] * _BN).reshape(1, HID) for l in range(4)]
    be = [params[f'be_c{l}'].reshape(1, HID) for l in range(4)]

    h, a_tab, b_tab = _tc_encode(
        x, params['W_lc1'], params['b_lc1'].reshape(1, HID),
        params['W_lc2'], params['b_lc2'].reshape(1, HID),
        wa[0], ba[0], wb[0])

    zc = jnp.zeros((N, CW), jnp.float32)
    ones = jnp.ones((CH, CW), jnp.float32)
    (cnt2,) = _sc_count(row.reshape(NROWS2D, CH), zc, ones)

    zc32 = jnp.zeros((N, DH), jnp.float32)
    (agg2,) = _sc_edge(a_tab, b_tab, row, col, zc32)

    for l in range(1, 4):
        h, a_tab, b_tab = _tc_combine(agg2, cnt2, h,
                                      sg[l - 1], be[l - 1],
                                      wa[l], ba[l], wb[l])
        (agg2,) = _sc_edge(a_tab, b_tab, row, col, zc32)

    o = _tc_final(agg2, cnt2, h, sg[3], be[3],
                  params['W_o1'], params['b_o1'].reshape(1, 64),
                  params['W_o2'], params['b_o2'].reshape(1, 32),
                  params['W_o3'], params['b_o3'].reshape(1, 8))
    return (o, batch)
